# baseline (speedup 1.0000x reference)
import functools
import jax
import jax.numpy as jnp
from jax import lax
from jax.experimental import pallas as pl
from jax.experimental.pallas import tpu as pltpu
from jax.experimental.pallas import tpu_sc as plsc

N_VOX = 48
VOX = 0.04
NPTS = 96000
NFEAT = 131072
C_DIM = 96
ROW_BLK = 768

N_SLABS = 12
SLAB_ROWS = 4 * N_VOX * N_VOX
SLAB_ALLOC = SLAB_ROWS + 256
DUMP_OFF = SLAB_ROWS + 16
PT_ALLOC = NPTS + 16
DUMP_ID = NPTS

NTILES = 16
PPT = NFEAT // NTILES
QPT = NPTS // NTILES
ZPT = SLAB_ALLOC // NTILES


IROWS = (PPT + 128) // 128 + 1
QROWS = (QPT + 128) // 128 + 1
TRASH_A = (IROWS - 1) * 128
TRASH_Q = (QROWS - 1) * 128


def _slab_body(feat_hbm, ucf_hbm, cf_hbm, csort_hbm, fval_hbm,
               pt_hbm, s_hbm,
               ucf_v, cf_v, cs_v, fv_v,
               idbuf, offbuf, aval,
               zrow, zline, sbuf, rowbuf, slab_sh, mslab_sh):
    qid = idbuf
    qoff = offbuf
    aoff = offbuf
    core = lax.axis_index("c")
    tile = lax.axis_index("s")
    pbase = tile * PPT
    qbase = tile * QPT

    pltpu.sync_copy(ucf_hbm.at[pl.ds(pbase, PPT)], ucf_v)
    pltpu.sync_copy(cf_hbm.at[pl.ds(qbase, QPT)], cf_v)
    pltpu.sync_copy(csort_hbm.at[pl.ds(qbase, QPT)], cs_v)
    pltpu.sync_copy(fval_hbm.at[pl.ds(qbase, QPT)], fv_v)

    zero16 = jnp.zeros((16,), jnp.float32)
    for i in range(16):
        for k in range(C_DIM // 16):
            zrow[i, pl.ds(k * 16, 16)] = zero16
    def _zl(i, _):
        zline[pl.ds(i * 16, 16)] = zero16
        return 0
    lax.fori_loop(0, ZPT // 16, _zl, 0)

    iota16 = lax.iota(jnp.int32, 16)

    def _scat2(ref, lin, vals):
        plsc.store_scatter(ref, [lax.shift_right_logical(lin, 7), lin & 127], vals)

    for p in range(N_SLABS // 2):
        sx = 2 * p + core
        base = sx * SLAB_ROWS

        def _zero(i, _):
            pltpu.sync_copy(zrow, slab_sh.at[pl.ds(tile * ZPT + i * 16, 16)])
            return 0
        lax.fori_loop(0, ZPT // 16, _zero, 0)
        pltpu.sync_copy(zline, mslab_sh.at[pl.ds(tile * ZPT, ZPT)])
        plsc.subcore_barrier()

        def _compactA(v, cnt):
            u = ucf_v[pl.ds(v * 16, 16)]
            msk = (u >= base) & (u < base + SLAB_ROWS)
            mi = msk.astype(jnp.int32)
            tgt = jnp.where(msk, cnt + plsc.cumsum(mi) - 1, TRASH_A)
            pos = pbase + v * 16 + iota16
            _scat2(idbuf, tgt, pos)
            _scat2(offbuf, tgt, u - base)
            return cnt + jnp.sum(mi)
        cnt = lax.fori_loop(0, PPT // 16, _compactA, 0)
        for k in range(8):
            padpos = cnt + k * 16 + iota16
            _scat2(idbuf, padpos, jnp.zeros((16,), jnp.int32))
            _scat2(offbuf, padpos, jnp.full((16,), DUMP_OFF, jnp.int32))

        def _chunkA(j, _):
            pltpu.sync_copy(feat_hbm.at[idbuf.at[j]], rowbuf)
            pltpu.sync_copy(rowbuf, slab_sh.at[offbuf.at[j]], add=True)
            return 0
        lax.fori_loop(0, (cnt + 127) // 128, _chunkA, 0)

        def _compactM(v, cnt):
            cvals = cs_v[pl.ds(v * 16, 16)]
            msk = (cvals >= base) & (cvals < base + SLAB_ROWS)
            mi = msk.astype(jnp.int32)
            tgt = jnp.where(msk, cnt + plsc.cumsum(mi) - 1, TRASH_Q)
            fvals = fv_v[pl.ds(v * 16, 16)]
            _scat2(aoff, tgt, cvals - base)
            plsc.store_scatter(aval, [tgt], fvals)
            return cnt + jnp.sum(mi)
        mcnt = lax.fori_loop(0, QPT // 16, _compactM, 0)
        for k in range(8):
            padpos = mcnt + k * 16 + iota16
            _scat2(aoff, padpos, jnp.full((16,), DUMP_OFF, jnp.int32))
            plsc.store_scatter(aval, [padpos], jnp.zeros((16,), jnp.float32))

        def _chunkM(j, _):
            pltpu.sync_copy(aval.at[pl.ds(j * 128, 128)], mslab_sh.at[aoff.at[j]], add=True)
            return 0
        lax.fori_loop(0, (mcnt + 127) // 128, _chunkM, 0)
        plsc.subcore_barrier()

        def _compactQ(v, cnt):
            cvals = cf_v[pl.ds(v * 16, 16)]
            msk = (cvals >= base) & (cvals < base + SLAB_ROWS)
            mi = msk.astype(jnp.int32)
            tgt = jnp.where(msk, cnt + plsc.cumsum(mi) - 1, TRASH_Q)
            pos = qbase + v * 16 + iota16
            _scat2(qid, tgt, pos)
            _scat2(qoff, tgt, cvals - base)
            return cnt + jnp.sum(mi)
        qcnt = lax.fori_loop(0, QPT // 16, _compactQ, 0)
        for k in range(8):
            padpos = qcnt + k * 16 + iota16
            _scat2(qid, padpos, jnp.full((16,), DUMP_ID, jnp.int32))
            _scat2(qoff, padpos, jnp.full((16,), DUMP_OFF, jnp.int32))

        def _chunkQ(j, _):
            pltpu.sync_copy(slab_sh.at[qoff.at[j]], rowbuf)
            pltpu.sync_copy(rowbuf, pt_hbm.at[qid.at[j]])
            pltpu.sync_copy(mslab_sh.at[qoff.at[j]], sbuf)
            pltpu.sync_copy(sbuf, s_hbm.at[qid.at[j]])
            return 0
        lax.fori_loop(0, (qcnt + 127) // 128, _chunkQ, 0)
        plsc.subcore_barrier()


def _make_slab_kernel():
    mesh = plsc.VectorSubcoreMesh(core_axis_name="c", subcore_axis_name="s")
    return pl.kernel(
        _slab_body,
        compiler_params=pltpu.CompilerParams(
            needs_layout_passes=False, use_tc_tiling_on_sc=False),
        out_type=(jax.ShapeDtypeStruct((PT_ALLOC, C_DIM), jnp.float32),
                  jax.ShapeDtypeStruct((PT_ALLOC,), jnp.float32)),
        mesh=mesh,
        scratch_types=[
            pltpu.VMEM((PPT,), jnp.int32),
            pltpu.VMEM((QPT,), jnp.int32),
            pltpu.VMEM((QPT,), jnp.int32),
            pltpu.VMEM((QPT,), jnp.float32),
            pltpu.VMEM((IROWS, 128), jnp.int32),
            pltpu.VMEM((IROWS, 128), jnp.int32),
            pltpu.VMEM((QROWS * 128,), jnp.float32),
            pltpu.VMEM((16, C_DIM), jnp.float32),
            pltpu.VMEM((ZPT,), jnp.float32),
            pltpu.VMEM((128,), jnp.float32),
            pltpu.VMEM((128, C_DIM), jnp.float32),
            pltpu.VMEM_SHARED((SLAB_ALLOC, C_DIM), jnp.float32),
            pltpu.VMEM_SHARED((SLAB_ALLOC,), jnp.float32),
        ],
    )


def _mlp_body(pt_ref, s_ref, w1_ref, b1_ref, w2_ref, b2_ref, w3_ref, b3_ref, out_ref):
    xb = pt_ref[...] * s_ref[...]
    h1 = jnp.maximum(jnp.dot(xb, w1_ref[...], preferred_element_type=jnp.float32) + b1_ref[...], 0.0)
    h2 = jnp.maximum(jnp.dot(h1, w2_ref[...], preferred_element_type=jnp.float32) + b2_ref[...], 0.0)
    out_ref[...] = jnp.dot(h2, w3_ref[...], preferred_element_type=jnp.float32) + b3_ref[...]


def _mlp(pt, s, W1, b1, W2, b2, W3, b3):
    C, H = W1.shape
    Co = W3.shape[1]
    n = pt.shape[0]
    return pl.pallas_call(
        _mlp_body,
        grid=(n // ROW_BLK,),
        in_specs=[
            pl.BlockSpec((ROW_BLK, C), lambda i: (i, 0)),
            pl.BlockSpec((ROW_BLK, 1), lambda i: (i, 0)),
            pl.BlockSpec((C, H), lambda i: (0, 0)),
            pl.BlockSpec((1, H), lambda i: (0, 0)),
            pl.BlockSpec((H, H), lambda i: (0, 0)),
            pl.BlockSpec((1, H), lambda i: (0, 0)),
            pl.BlockSpec((H, Co), lambda i: (0, 0)),
            pl.BlockSpec((1, Co), lambda i: (0, 0)),
        ],
        out_specs=pl.BlockSpec((ROW_BLK, Co), lambda i: (i, 0)),
        out_shape=jax.ShapeDtypeStruct((n, Co), jnp.float32),
    )(pt, s, W1, b1.reshape(1, H), W2, b2.reshape(1, H), W3, b3.reshape(1, Co))


def kernel(depth, img_size, rotmats, tvecs, K, feat, up_coords, interval, origin, W1, b1, W2, b2, W3, b3):
    d = depth[2:-2]
    V, h, w = d.shape
    Himg = img_size[0].astype(jnp.float32)
    Wimg = img_size[1].astype(jnp.float32)
    K_inv = jnp.linalg.inv(K[2:-2])
    R_T = jnp.swapaxes(rotmats[2:-2], 1, 2)
    xs = (jnp.arange(w, dtype=jnp.float32) + 0.5) * (Wimg / w)
    ys = (jnp.arange(h, dtype=jnp.float32) + 0.5) * (Himg / h)
    vv, uu = jnp.meshgrid(ys, xs, indexing='ij')
    homo = jnp.stack([uu.reshape(-1), vv.reshape(-1), jnp.ones(h * w, jnp.float32)], axis=0)
    homo = jnp.broadcast_to(homo, (V, 3, h * w))
    pig = homo * d.reshape(V, 1, -1)
    cam = jnp.matmul(K_inv, pig) - tvecs[2:-2][:, :, None]
    world = jnp.matmul(R_T, cam)
    pts = jnp.swapaxes(world, 1, 2).reshape(-1, 3)
    idx3d = jnp.floor((pts - origin) / VOX).astype(jnp.int32)
    x, y, z = idx3d[:, 0], idx3d[:, 1], idx3d[:, 2]

    perm = jnp.lexsort((z, y, x))
    sx, sy, sz = x[perm], y[perm], z[perm]
    first = jnp.concatenate([
        jnp.ones((1,), jnp.float32),
        ((sx[1:] != sx[:-1]) | (sy[1:] != sy[:-1]) | (sz[1:] != sz[:-1])).astype(jnp.float32)])
    csort = (jnp.clip(sx, 0, N_VOX - 1) * N_VOX + jnp.clip(sy, 0, N_VOX - 1)) * N_VOX + jnp.clip(sz, 0, N_VOX - 1)

    uc = jnp.clip(jnp.round(up_coords[:, 1:4] / interval[0]).astype(jnp.int32), 0, N_VOX - 1)
    ucf = (uc[:, 0] * N_VOX + uc[:, 1]) * N_VOX + uc[:, 2]
    cf = (jnp.clip(x, 0, N_VOX - 1) * N_VOX + jnp.clip(y, 0, N_VOX - 1)) * N_VOX + jnp.clip(z, 0, N_VOX - 1)

    pt_pad, s_pad = _make_slab_kernel()(feat, ucf, cf, csort, first)
    pt = pt_pad[:NPTS]
    s = (1.0 + s_pad[:NPTS]).reshape(-1, 1)
    return _mlp(pt, s, W1, b1, W2, b2, W3, b3)


# speedup vs baseline: 1.4544x; 1.4544x over previous
import functools
import jax
import jax.numpy as jnp
from jax import lax
from jax.experimental import pallas as pl
from jax.experimental.pallas import tpu as pltpu
from jax.experimental.pallas import tpu_sc as plsc

N_VOX = 48
VOX = 0.04
NPTS = 96000
NFEAT = 131072
C_DIM = 96
ROW_BLK = 768

N_SLABS = 12
SLAB_ROWS = 4 * N_VOX * N_VOX
SLAB_ALLOC = SLAB_ROWS + 256
DUMP_OFF = SLAB_ROWS + 16
PT_ALLOC = NPTS + 16
DUMP_ID = NPTS

NTILES = 16
PPT = NFEAT // NTILES
QPT = NPTS // NTILES
ZPT = SLAB_ALLOC // NTILES


def _slab_body(feat_hbm, ucf_hbm, cf_hbm, csort_hbm, fval_hbm,
               pt_hbm, s_hbm,
               ucf_v, cf_v, cs_v, fv_v,
               idbuf, offbuf, aval,
               zrow, zline, sbuf, rowbuf, slab_sh, mslab_sh):
    qid = idbuf
    qoff = offbuf
    aoff = offbuf
    core = lax.axis_index("c")
    tile = lax.axis_index("s")
    pbase = tile * PPT
    qbase = tile * QPT

    pltpu.sync_copy(ucf_hbm.at[pl.ds(pbase, PPT)], ucf_v)
    pltpu.sync_copy(cf_hbm.at[pl.ds(qbase, QPT)], cf_v)
    pltpu.sync_copy(csort_hbm.at[pl.ds(qbase, QPT)], cs_v)
    pltpu.sync_copy(fval_hbm.at[pl.ds(qbase, QPT)], fv_v)

    zero16 = jnp.zeros((16,), jnp.float32)
    for i in range(16):
        for k in range(C_DIM // 16):
            zrow[i, pl.ds(k * 16, 16)] = zero16
    def _zl(i, _):
        zline[pl.ds(i * 16, 16)] = zero16
        return 0
    lax.fori_loop(0, ZPT // 16, _zl, 0)

    iota16 = lax.iota(jnp.int32, 16)

    for p in range(N_SLABS // 2):
        sx = 2 * p + core
        base = sx * SLAB_ROWS

        def _zero(i, _):
            pltpu.sync_copy(zrow, slab_sh.at[pl.ds(tile * ZPT + i * 16, 16)])
            return 0
        lax.fori_loop(0, ZPT // 16, _zero, 0)
        pltpu.sync_copy(zline, mslab_sh.at[pl.ds(tile * ZPT, ZPT)])
        plsc.subcore_barrier()

        def _compactA(v, cnt):
            u = ucf_v[pl.ds(v * 16, 16)]
            msk = (u >= base) & (u < base + SLAB_ROWS)
            mi = msk.astype(jnp.int32)
            tgt = jnp.where(msk, cnt + plsc.cumsum(mi) - 1, PPT + 16)
            pos = pbase + v * 16 + iota16
            plsc.store_scatter(idbuf, [tgt], pos)
            plsc.store_scatter(offbuf, [tgt], u - base)
            return cnt + jnp.sum(mi)
        cnt = lax.fori_loop(0, PPT // 16, _compactA, 0)
        idbuf[pl.ds(cnt, 16)] = jnp.zeros((16,), jnp.int32)
        offbuf[pl.ds(cnt, 16)] = jnp.full((16,), DUMP_OFF, jnp.int32)

        def _chunkA(i, _):
            ids = idbuf[pl.ds(i * 16, 16)]
            offs = offbuf[pl.ds(i * 16, 16)]
            pltpu.sync_copy(feat_hbm.at[ids], rowbuf)
            pltpu.sync_copy(rowbuf, slab_sh.at[offs], add=True)
            return 0
        lax.fori_loop(0, (cnt + 15) // 16, _chunkA, 0)

        def _compactM(v, cnt):
            cvals = cs_v[pl.ds(v * 16, 16)]
            msk = (cvals >= base) & (cvals < base + SLAB_ROWS)
            mi = msk.astype(jnp.int32)
            tgt = jnp.where(msk, cnt + plsc.cumsum(mi) - 1, QPT + 16)
            fvals = fv_v[pl.ds(v * 16, 16)]
            plsc.store_scatter(aoff, [tgt], cvals - base)
            plsc.store_scatter(aval, [tgt], fvals)
            return cnt + jnp.sum(mi)
        mcnt = lax.fori_loop(0, QPT // 16, _compactM, 0)
        aoff[pl.ds(mcnt, 16)] = jnp.full((16,), DUMP_OFF, jnp.int32)
        aval[pl.ds(mcnt, 16)] = jnp.zeros((16,), jnp.float32)

        def _chunkM(i, _):
            offs = aoff[pl.ds(i * 16, 16)]
            pltpu.sync_copy(aval.at[pl.ds(i * 16, 16)], mslab_sh.at[offs], add=True)
            return 0
        lax.fori_loop(0, (mcnt + 15) // 16, _chunkM, 0)
        plsc.subcore_barrier()

        def _compactQ(v, cnt):
            cvals = cf_v[pl.ds(v * 16, 16)]
            msk = (cvals >= base) & (cvals < base + SLAB_ROWS)
            mi = msk.astype(jnp.int32)
            tgt = jnp.where(msk, cnt + plsc.cumsum(mi) - 1, QPT + 16)
            pos = qbase + v * 16 + iota16
            plsc.store_scatter(qid, [tgt], pos)
            plsc.store_scatter(qoff, [tgt], cvals - base)
            return cnt + jnp.sum(mi)
        qcnt = lax.fori_loop(0, QPT // 16, _compactQ, 0)
        qid[pl.ds(qcnt, 16)] = jnp.full((16,), DUMP_ID, jnp.int32)
        qoff[pl.ds(qcnt, 16)] = jnp.full((16,), DUMP_OFF, jnp.int32)

        def _chunkQ(i, _):
            ids = qid[pl.ds(i * 16, 16)]
            offs = qoff[pl.ds(i * 16, 16)]
            pltpu.sync_copy(slab_sh.at[offs], rowbuf)
            pltpu.sync_copy(rowbuf, pt_hbm.at[ids])
            pltpu.sync_copy(mslab_sh.at[offs], sbuf)
            pltpu.sync_copy(sbuf, s_hbm.at[ids])
            return 0
        lax.fori_loop(0, (qcnt + 15) // 16, _chunkQ, 0)
        plsc.subcore_barrier()


def _make_slab_kernel():
    mesh = plsc.VectorSubcoreMesh(core_axis_name="c", subcore_axis_name="s")
    return pl.kernel(
        _slab_body,
        compiler_params=pltpu.CompilerParams(
            needs_layout_passes=False, use_tc_tiling_on_sc=False),
        out_type=(jax.ShapeDtypeStruct((PT_ALLOC, C_DIM), jnp.float32),
                  jax.ShapeDtypeStruct((PT_ALLOC,), jnp.float32)),
        mesh=mesh,
        scratch_types=[
            pltpu.VMEM((PPT,), jnp.int32),
            pltpu.VMEM((QPT,), jnp.int32),
            pltpu.VMEM((QPT,), jnp.int32),
            pltpu.VMEM((QPT,), jnp.float32),
            pltpu.VMEM((PPT + 32,), jnp.int32),
            pltpu.VMEM((PPT + 32,), jnp.int32),
            pltpu.VMEM((QPT + 32,), jnp.float32),
            pltpu.VMEM((16, C_DIM), jnp.float32),
            pltpu.VMEM((ZPT,), jnp.float32),
            pltpu.VMEM((16,), jnp.float32),
            pltpu.VMEM((16, C_DIM), jnp.float32),
            pltpu.VMEM_SHARED((SLAB_ALLOC, C_DIM), jnp.float32),
            pltpu.VMEM_SHARED((SLAB_ALLOC,), jnp.float32),
        ],
    )


def _mlp_body(pt_ref, s_ref, w1_ref, b1_ref, w2_ref, b2_ref, w3_ref, b3_ref, out_ref):
    xb = pt_ref[...] * s_ref[...]
    h1 = jnp.maximum(jnp.dot(xb, w1_ref[...], preferred_element_type=jnp.float32) + b1_ref[...], 0.0)
    h2 = jnp.maximum(jnp.dot(h1, w2_ref[...], preferred_element_type=jnp.float32) + b2_ref[...], 0.0)
    out_ref[...] = jnp.dot(h2, w3_ref[...], preferred_element_type=jnp.float32) + b3_ref[...]


def _mlp(pt, s, W1, b1, W2, b2, W3, b3):
    C, H = W1.shape
    Co = W3.shape[1]
    n = pt.shape[0]
    return pl.pallas_call(
        _mlp_body,
        grid=(n // ROW_BLK,),
        in_specs=[
            pl.BlockSpec((ROW_BLK, C), lambda i: (i, 0)),
            pl.BlockSpec((ROW_BLK, 1), lambda i: (i, 0)),
            pl.BlockSpec((C, H), lambda i: (0, 0)),
            pl.BlockSpec((1, H), lambda i: (0, 0)),
            pl.BlockSpec((H, H), lambda i: (0, 0)),
            pl.BlockSpec((1, H), lambda i: (0, 0)),
            pl.BlockSpec((H, Co), lambda i: (0, 0)),
            pl.BlockSpec((1, Co), lambda i: (0, 0)),
        ],
        out_specs=pl.BlockSpec((ROW_BLK, Co), lambda i: (i, 0)),
        out_shape=jax.ShapeDtypeStruct((n, Co), jnp.float32),
    )(pt, s, W1, b1.reshape(1, H), W2, b2.reshape(1, H), W3, b3.reshape(1, Co))


def kernel(depth, img_size, rotmats, tvecs, K, feat, up_coords, interval, origin, W1, b1, W2, b2, W3, b3):
    d = depth[2:-2]
    V, h, w = d.shape
    Himg = img_size[0].astype(jnp.float32)
    Wimg = img_size[1].astype(jnp.float32)
    K_inv = jnp.linalg.inv(K[2:-2])
    R_T = jnp.swapaxes(rotmats[2:-2], 1, 2)
    xs = (jnp.arange(w, dtype=jnp.float32) + 0.5) * (Wimg / w)
    ys = (jnp.arange(h, dtype=jnp.float32) + 0.5) * (Himg / h)
    vv, uu = jnp.meshgrid(ys, xs, indexing='ij')
    homo = jnp.stack([uu.reshape(-1), vv.reshape(-1), jnp.ones(h * w, jnp.float32)], axis=0)
    homo = jnp.broadcast_to(homo, (V, 3, h * w))
    pig = homo * d.reshape(V, 1, -1)
    cam = jnp.matmul(K_inv, pig) - tvecs[2:-2][:, :, None]
    world = jnp.matmul(R_T, cam)
    pts = jnp.swapaxes(world, 1, 2).reshape(-1, 3)
    idx3d = jnp.floor((pts - origin) / VOX).astype(jnp.int32)
    x, y, z = idx3d[:, 0], idx3d[:, 1], idx3d[:, 2]

    perm = jnp.lexsort((z, y, x))
    sx, sy, sz = x[perm], y[perm], z[perm]
    first = jnp.concatenate([
        jnp.ones((1,), jnp.float32),
        ((sx[1:] != sx[:-1]) | (sy[1:] != sy[:-1]) | (sz[1:] != sz[:-1])).astype(jnp.float32)])
    csort = (jnp.clip(sx, 0, N_VOX - 1) * N_VOX + jnp.clip(sy, 0, N_VOX - 1)) * N_VOX + jnp.clip(sz, 0, N_VOX - 1)

    uc = jnp.clip(jnp.round(up_coords[:, 1:4] / interval[0]).astype(jnp.int32), 0, N_VOX - 1)
    ucf = (uc[:, 0] * N_VOX + uc[:, 1]) * N_VOX + uc[:, 2]
    cf = (jnp.clip(x, 0, N_VOX - 1) * N_VOX + jnp.clip(y, 0, N_VOX - 1)) * N_VOX + jnp.clip(z, 0, N_VOX - 1)

    pt_pad, s_pad = _make_slab_kernel()(feat, ucf, cf, csort, first)
    pt = pt_pad[:NPTS]
    s = (1.0 + s_pad[:NPTS]).reshape(-1, 1)
    return _mlp(pt, s, W1, b1, W2, b2, W3, b3)


# speedup vs baseline: 1.6632x; 1.1436x over previous
import functools
import jax
import jax.numpy as jnp
from jax import lax
from jax.experimental import pallas as pl
from jax.experimental.pallas import tpu as pltpu
from jax.experimental.pallas import tpu_sc as plsc

N_VOX = 48
VOX = 0.04
NPTS = 96000
NFEAT = 131072
C_DIM = 96
ROW_BLK = 768

N_SLABS = 12
SLAB_ROWS = 4 * N_VOX * N_VOX
SLAB_ALLOC = SLAB_ROWS + 256
DUMP_OFF = SLAB_ROWS + 16
PT_ALLOC = NPTS + 16
DUMP_ID = NPTS

NTILES = 16
PPT = NFEAT // NTILES
QPT = NPTS // NTILES
ZPT = SLAB_ALLOC // NTILES


def _slab_body(feat_hbm, ucf_hbm, cf_hbm,
               pt_hbm,
               ucf_v, cf_v,
               idbuf, offbuf,
               zrow, rowbuf, slab_sh):
    qid = idbuf
    qoff = offbuf
    core = lax.axis_index("c")
    tile = lax.axis_index("s")
    pbase = tile * PPT
    qbase = tile * QPT

    pltpu.sync_copy(ucf_hbm.at[pl.ds(pbase, PPT)], ucf_v)
    pltpu.sync_copy(cf_hbm.at[pl.ds(qbase, QPT)], cf_v)

    zero16 = jnp.zeros((16,), jnp.float32)
    for i in range(16):
        for k in range(C_DIM // 16):
            zrow[i, pl.ds(k * 16, 16)] = zero16
    iota16 = lax.iota(jnp.int32, 16)

    for p in range(N_SLABS // 2):
        sx = 2 * p + core
        base = sx * SLAB_ROWS

        def _zero(i, _):
            pltpu.sync_copy(zrow, slab_sh.at[pl.ds(tile * ZPT + i * 16, 16)])
            return 0
        lax.fori_loop(0, ZPT // 16, _zero, 0)
        plsc.subcore_barrier()

        def _compactA(v, cnt):
            u = ucf_v[pl.ds(v * 16, 16)]
            msk = (u >= base) & (u < base + SLAB_ROWS)
            mi = msk.astype(jnp.int32)
            tgt = jnp.where(msk, cnt + plsc.cumsum(mi) - 1, PPT + 16)
            pos = pbase + v * 16 + iota16
            plsc.store_scatter(idbuf, [tgt], pos)
            plsc.store_scatter(offbuf, [tgt], u - base)
            return cnt + jnp.sum(mi)
        cnt = lax.fori_loop(0, PPT // 16, _compactA, 0)
        idbuf[pl.ds(cnt, 16)] = jnp.zeros((16,), jnp.int32)
        offbuf[pl.ds(cnt, 16)] = jnp.full((16,), DUMP_OFF, jnp.int32)

        def _chunkA(i, _):
            ids = idbuf[pl.ds(i * 16, 16)]
            offs = offbuf[pl.ds(i * 16, 16)]
            pltpu.sync_copy(feat_hbm.at[ids], rowbuf)
            pltpu.sync_copy(rowbuf, slab_sh.at[offs], add=True)
            return 0
        lax.fori_loop(0, (cnt + 15) // 16, _chunkA, 0)

        def _compactQ(v, cnt):
            cvals = cf_v[pl.ds(v * 16, 16)]
            msk = (cvals >= base) & (cvals < base + SLAB_ROWS)
            mi = msk.astype(jnp.int32)
            tgt = jnp.where(msk, cnt + plsc.cumsum(mi) - 1, QPT + 16)
            pos = qbase + v * 16 + iota16
            plsc.store_scatter(qid, [tgt], pos)
            plsc.store_scatter(qoff, [tgt], cvals - base)
            return cnt + jnp.sum(mi)
        qcnt = lax.fori_loop(0, QPT // 16, _compactQ, 0)
        qid[pl.ds(qcnt, 16)] = jnp.full((16,), DUMP_ID, jnp.int32)
        qoff[pl.ds(qcnt, 16)] = jnp.full((16,), DUMP_OFF, jnp.int32)

        def _chunkQ(i, _):
            ids = qid[pl.ds(i * 16, 16)]
            offs = qoff[pl.ds(i * 16, 16)]
            pltpu.sync_copy(slab_sh.at[offs], rowbuf)
            pltpu.sync_copy(rowbuf, pt_hbm.at[ids])
            return 0
        lax.fori_loop(0, (qcnt + 15) // 16, _chunkQ, 0)
        plsc.subcore_barrier()


def _make_slab_kernel():
    mesh = plsc.VectorSubcoreMesh(core_axis_name="c", subcore_axis_name="s")
    return pl.kernel(
        _slab_body,
        compiler_params=pltpu.CompilerParams(
            needs_layout_passes=False, use_tc_tiling_on_sc=False),
        out_type=jax.ShapeDtypeStruct((PT_ALLOC, C_DIM), jnp.float32),
        mesh=mesh,
        scratch_types=[
            pltpu.VMEM((PPT,), jnp.int32),
            pltpu.VMEM((QPT,), jnp.int32),
            pltpu.VMEM((PPT + 32,), jnp.int32),
            pltpu.VMEM((PPT + 32,), jnp.int32),
            pltpu.VMEM((16, C_DIM), jnp.float32),
            pltpu.VMEM((16, C_DIM), jnp.float32),
            pltpu.VMEM_SHARED((SLAB_ALLOC, C_DIM), jnp.float32),
        ],
    )


def _mlp_body(pt_ref, s_ref, w1_ref, b1_ref, w2_ref, b2_ref, w3_ref, b3_ref, out_ref):
    xb = pt_ref[...] * s_ref[...]
    h1 = jnp.maximum(jnp.dot(xb, w1_ref[...], preferred_element_type=jnp.float32) + b1_ref[...], 0.0)
    h2 = jnp.maximum(jnp.dot(h1, w2_ref[...], preferred_element_type=jnp.float32) + b2_ref[...], 0.0)
    out_ref[...] = jnp.dot(h2, w3_ref[...], preferred_element_type=jnp.float32) + b3_ref[...]


def _mlp(pt, s, W1, b1, W2, b2, W3, b3):
    C, H = W1.shape
    Co = W3.shape[1]
    n = pt.shape[0]
    return pl.pallas_call(
        _mlp_body,
        grid=(n // ROW_BLK,),
        in_specs=[
            pl.BlockSpec((ROW_BLK, C), lambda i: (i, 0)),
            pl.BlockSpec((ROW_BLK, 1), lambda i: (i, 0)),
            pl.BlockSpec((C, H), lambda i: (0, 0)),
            pl.BlockSpec((1, H), lambda i: (0, 0)),
            pl.BlockSpec((H, H), lambda i: (0, 0)),
            pl.BlockSpec((1, H), lambda i: (0, 0)),
            pl.BlockSpec((H, Co), lambda i: (0, 0)),
            pl.BlockSpec((1, Co), lambda i: (0, 0)),
        ],
        out_specs=pl.BlockSpec((ROW_BLK, Co), lambda i: (i, 0)),
        out_shape=jax.ShapeDtypeStruct((n, Co), jnp.float32),
    )(pt, s, W1, b1.reshape(1, H), W2, b2.reshape(1, H), W3, b3.reshape(1, Co))


def kernel(depth, img_size, rotmats, tvecs, K, feat, up_coords, interval, origin, W1, b1, W2, b2, W3, b3):
    d = depth[2:-2]
    V, h, w = d.shape
    Himg = img_size[0].astype(jnp.float32)
    Wimg = img_size[1].astype(jnp.float32)
    K_inv = jnp.linalg.inv(K[2:-2])
    R_T = jnp.swapaxes(rotmats[2:-2], 1, 2)
    xs = (jnp.arange(w, dtype=jnp.float32) + 0.5) * (Wimg / w)
    ys = (jnp.arange(h, dtype=jnp.float32) + 0.5) * (Himg / h)
    vv, uu = jnp.meshgrid(ys, xs, indexing='ij')
    homo = jnp.stack([uu.reshape(-1), vv.reshape(-1), jnp.ones(h * w, jnp.float32)], axis=0)
    homo = jnp.broadcast_to(homo, (V, 3, h * w))
    pig = homo * d.reshape(V, 1, -1)
    cam = jnp.matmul(K_inv, pig) - tvecs[2:-2][:, :, None]
    world = jnp.matmul(R_T, cam)
    pts = jnp.swapaxes(world, 1, 2).reshape(-1, 3)
    idx3d = jnp.floor((pts - origin) / VOX).astype(jnp.int32)
    x, y, z = idx3d[:, 0], idx3d[:, 1], idx3d[:, 2]

    perm = jnp.lexsort((z, y, x))
    sx, sy, sz = x[perm], y[perm], z[perm]
    first = jnp.concatenate([
        jnp.ones((1,), jnp.float32),
        ((sx[1:] != sx[:-1]) | (sy[1:] != sy[:-1]) | (sz[1:] != sz[:-1])).astype(jnp.float32)])
    csort = (jnp.clip(sx, 0, N_VOX - 1) * N_VOX + jnp.clip(sy, 0, N_VOX - 1)) * N_VOX + jnp.clip(sz, 0, N_VOX - 1)
    m = jnp.zeros((N_VOX ** 3,), jnp.float32).at[csort].add(first)

    uc = jnp.clip(jnp.round(up_coords[:, 1:4] / interval[0]).astype(jnp.int32), 0, N_VOX - 1)
    ucf = (uc[:, 0] * N_VOX + uc[:, 1]) * N_VOX + uc[:, 2]
    cf = (jnp.clip(x, 0, N_VOX - 1) * N_VOX + jnp.clip(y, 0, N_VOX - 1)) * N_VOX + jnp.clip(z, 0, N_VOX - 1)

    pt_pad = _make_slab_kernel()(feat, ucf, cf)
    pt = pt_pad[:NPTS]
    s = (1.0 + m[cf]).reshape(-1, 1)
    return _mlp(pt, s, W1, b1, W2, b2, W3, b3)
